# Initial kernel scaffold; baseline (speedup 1.0000x reference)
import functools

import jax
import jax.numpy as jnp
from jax import lax
from jax.experimental import pallas as pl
from jax.experimental.pallas import tpu as pltpu
from jax.experimental.pallas import tpu_sc as plsc

_L = 16


def _make_gather(B, D, NC, NS):
    NW = NC * NS
    per_w = B // NW
    C = 1600
    n_chunks = per_w // C

    mesh = plsc.VectorSubcoreMesh(core_axis_name="c", subcore_axis_name="s")

    @functools.partial(
        pl.kernel,
        mesh=mesh,
        out_type=jax.ShapeDtypeStruct((B, D), jnp.float32),
        scratch_types=[
            pltpu.VMEM((per_w,), jnp.int32),
            pltpu.VMEM((C, D), jnp.float32),
            pltpu.SemaphoreType.DMA,
        ],
    )
    def gather_kernel(table_hbm, idx_hbm, out_hbm, idx_v, rows_v, sem):
        wid = lax.axis_index("s") * NC + lax.axis_index("c")
        base = wid * per_w
        pltpu.sync_copy(idx_hbm.at[pl.ds(base, per_w)], idx_v)

        def fix(i, c):
            sl = pl.ds(i * _L, _L)
            idx_v[sl] = idx_v[sl] & 63
            return c

        lax.fori_loop(0, per_w // _L, fix, 0)

        def chunk(g, c):
            off = g * C
            pltpu.async_copy(
                table_hbm.at[idx_v.at[pl.ds(off, C)]], rows_v, sem
            ).wait()
            pltpu.sync_copy(rows_v, out_hbm.at[pl.ds(base + off, C)])
            return c

        lax.fori_loop(0, n_chunks, chunk, 0)

    return gather_kernel


def kernel(embedding, sequence_ids):
    Bt, S = sequence_ids.shape
    V, D = embedding.shape
    table = jnp.zeros((64, D), jnp.float32).at[:V].set(embedding)
    ids = sequence_ids.reshape(-1).astype(jnp.int32)
    info = plsc.get_sparse_core_info()
    out = _make_gather(ids.shape[0], D, info.num_cores, info.num_subcores)(
        table, ids
    )
    return out.reshape(Bt, S, D)


# speedup vs baseline: 1.7818x; 1.7818x over previous
import functools

import jax
import jax.numpy as jnp
from jax import lax
from jax.experimental import pallas as pl
from jax.experimental.pallas import tpu as pltpu
from jax.experimental.pallas import tpu_sc as plsc

_L = 16


def _make_gather(B, D, NC, NS):
    NW = NC * NS
    per_w = B // NW
    C = 1600
    n_chunks = per_w // C

    mesh = plsc.VectorSubcoreMesh(core_axis_name="c", subcore_axis_name="s")

    @functools.partial(
        pl.kernel,
        mesh=mesh,
        out_type=jax.ShapeDtypeStruct((B, D), jnp.float32),
        scratch_types=[
            pltpu.VMEM((per_w,), jnp.int32),
            pltpu.VMEM((C, D), jnp.float32),
            pltpu.SemaphoreType.DMA,
        ],
        compiler_params=pltpu.CompilerParams(use_tc_tiling_on_sc=False),
    )
    def gather_kernel(table_hbm, idx_hbm, out_hbm, idx_v, rows_v, sem):
        wid = lax.axis_index("s") * NC + lax.axis_index("c")
        base = wid * per_w
        pltpu.sync_copy(idx_hbm.at[pl.ds(base, per_w)], idx_v)

        def fix(i, c):
            sl = pl.ds(i * _L, _L)
            idx_v[sl] = idx_v[sl] & 63
            return c

        lax.fori_loop(0, per_w // _L, fix, 0)

        def chunk(g, c):
            off = g * C
            pltpu.async_copy(
                table_hbm.at[idx_v.at[pl.ds(off, C)]], rows_v, sem
            ).wait()
            pltpu.sync_copy(rows_v, out_hbm.at[pl.ds(base + off, C)])
            return c

        lax.fori_loop(0, n_chunks, chunk, 0)

    return gather_kernel


def kernel(embedding, sequence_ids):
    Bt, S = sequence_ids.shape
    V, D = embedding.shape
    table = jnp.zeros((64, D), jnp.float32).at[:V].set(embedding)
    ids = sequence_ids.reshape(-1).astype(jnp.int32)
    info = plsc.get_sparse_core_info()
    out = _make_gather(ids.shape[0], D, info.num_cores, info.num_subcores)(
        table, ids
    )
    return out.reshape(Bt, S, D)


# speedup vs baseline: 4.3224x; 2.4259x over previous
import functools

import jax
import jax.numpy as jnp
from jax import lax
from jax.experimental import pallas as pl
from jax.experimental.pallas import tpu as pltpu
from jax.experimental.pallas import tpu_sc as plsc

_L = 16


def _make_gather(B, NC, NS):
    NW = NC * NS
    pairs = B // 2
    per_w = pairs // NW
    P = 256
    n_chunks = per_w // P
    ids_per_w = per_w * 2

    mesh = plsc.VectorSubcoreMesh(core_axis_name="c", subcore_axis_name="s")

    @functools.partial(
        pl.kernel,
        mesh=mesh,
        out_type=jax.ShapeDtypeStruct((pairs, 128), jnp.float32),
        scratch_types=[
            pltpu.VMEM((ids_per_w,), jnp.int32),
            pltpu.VMEM((P,), jnp.int32),
            pltpu.VMEM((P,), jnp.int32),
            pltpu.VMEM((2, P, 128), jnp.float32),
            pltpu.SemaphoreType.DMA,
            pltpu.SemaphoreType.DMA,
            pltpu.SemaphoreType.DMA,
            pltpu.SemaphoreType.DMA,
        ],
        compiler_params=pltpu.CompilerParams(needs_layout_passes=False),
    )
    def gather_kernel(
        tbl_hbm, idx_hbm, out_hbm, idx_v, pidx0, pidx1, rows, sg0, sg1, ss0, ss1
    ):
        sg = [sg0, sg1]
        ss = [ss0, ss1]
        pidx = [pidx0, pidx1]
        wid = lax.axis_index("s") * NC + lax.axis_index("c")
        ibase = wid * ids_per_w
        obase = wid * per_w
        pltpu.sync_copy(idx_hbm.at[pl.ds(ibase, ids_per_w)], idx_v)
        lanes2 = lax.iota(jnp.int32, _L) * 2

        def build(g, b):
            cbase = g * (2 * P)

            def it(i, c):
                pos_a = cbase + i * (2 * _L) + lanes2
                a = plsc.load_gather(idx_v, [pos_a])
                bb = plsc.load_gather(idx_v, [pos_a + 1])
                pidx[b][pl.ds(i * _L, _L)] = (a & 63) * 64 + (bb & 63)
                return c

            lax.fori_loop(0, P // _L, it, 0)

        def start_gather(g, b):
            return pltpu.async_copy(tbl_hbm.at[pidx[b]], rows.at[b], sg[b])

        build(0, 0)
        gcopy = [start_gather(0, 0), None]
        scopy = [None, None]
        for g in range(n_chunks):
            b = g & 1
            nb = 1 - b
            if g + 1 < n_chunks:
                build(g + 1, nb)
            gcopy[b].wait()
            scopy[b] = pltpu.async_copy(
                rows.at[b], out_hbm.at[pl.ds(obase + g * P, P)], ss[b]
            )
            if g + 1 < n_chunks:
                if scopy[nb] is not None:
                    scopy[nb].wait()
                gcopy[nb] = start_gather(g + 1, nb)
        if n_chunks >= 2:
            scopy[(n_chunks - 2) & 1].wait()
        scopy[(n_chunks - 1) & 1].wait()

    return gather_kernel


def kernel(embedding, sequence_ids):
    Bt, S = sequence_ids.shape
    V, D = embedding.shape
    tbl = jnp.zeros((64, D), jnp.float32).at[:V].set(embedding)
    left = jnp.broadcast_to(tbl[:, None, :], (64, 64, D))
    right = jnp.broadcast_to(tbl[None, :, :], (64, 64, D))
    tbl_pairs = jnp.concatenate([left, right], axis=-1).reshape(64 * 64, 2 * D)
    ids = sequence_ids.reshape(-1).astype(jnp.int32)
    info = plsc.get_sparse_core_info()
    out = _make_gather(ids.shape[0], info.num_cores, info.num_subcores)(
        tbl_pairs, ids
    )
    return out.reshape(Bt, S, D)


# speedup vs baseline: 4.9928x; 1.1551x over previous
import functools

import jax
import jax.numpy as jnp
from jax import lax
from jax.experimental import pallas as pl
from jax.experimental.pallas import tpu as pltpu
from jax.experimental.pallas import tpu_sc as plsc

_L = 16


def _make_gather(B, NC, NS):
    NW = NC * NS
    pairs = B // 2
    per_w = pairs // NW
    P = 256
    n_chunks = per_w // P
    ids_per_w = per_w * 2

    mesh = plsc.VectorSubcoreMesh(core_axis_name="c", subcore_axis_name="s")

    @functools.partial(
        pl.kernel,
        mesh=mesh,
        out_type=jax.ShapeDtypeStruct((pairs, 128), jnp.float32),
        scratch_types=[
            pltpu.VMEM((ids_per_w,), jnp.int32),
            pltpu.VMEM((P,), jnp.int32),
            pltpu.VMEM((P,), jnp.int32),
            pltpu.VMEM((2, P, 128), jnp.float32),
            pltpu.VMEM_SHARED((4096, 128), jnp.float32),
            pltpu.SemaphoreType.DMA,
            pltpu.SemaphoreType.DMA,
            pltpu.SemaphoreType.DMA,
            pltpu.SemaphoreType.DMA,
        ],
        compiler_params=pltpu.CompilerParams(needs_layout_passes=False),
    )
    def gather_kernel(
        tbl_hbm, idx_hbm, out_hbm, idx_v, pidx0, pidx1, rows, tbl_sh,
        sg0, sg1, ss0, ss1
    ):
        sg = [sg0, sg1]
        ss = [ss0, ss1]
        pidx = [pidx0, pidx1]
        sid = lax.axis_index("s")
        wid = sid * NC + lax.axis_index("c")
        ibase = wid * ids_per_w
        obase = wid * per_w

        @pl.when(sid == 0)
        def _():
            pltpu.sync_copy(tbl_hbm, tbl_sh)

        pltpu.sync_copy(idx_hbm.at[pl.ds(ibase, ids_per_w)], idx_v)
        plsc.subcore_barrier()
        lanes2 = lax.iota(jnp.int32, _L) * 2

        def build(g, b):
            cbase = g * (2 * P)

            def it(i, c):
                pos_a = cbase + i * (2 * _L) + lanes2
                a = plsc.load_gather(idx_v, [pos_a])
                bb = plsc.load_gather(idx_v, [pos_a + 1])
                pidx[b][pl.ds(i * _L, _L)] = (a & 63) * 64 + (bb & 63)
                return c

            lax.fori_loop(0, P // _L, it, 0)

        def start_gather(g, b):
            return pltpu.async_copy(tbl_sh.at[pidx[b]], rows.at[b], sg[b])

        build(0, 0)
        gcopy = [start_gather(0, 0), None]
        scopy = [None, None]
        for g in range(n_chunks):
            b = g & 1
            nb = 1 - b
            if g + 1 < n_chunks:
                build(g + 1, nb)
            gcopy[b].wait()
            scopy[b] = pltpu.async_copy(
                rows.at[b], out_hbm.at[pl.ds(obase + g * P, P)], ss[b]
            )
            if g + 1 < n_chunks:
                if scopy[nb] is not None:
                    scopy[nb].wait()
                gcopy[nb] = start_gather(g + 1, nb)
        if n_chunks >= 2:
            scopy[(n_chunks - 2) & 1].wait()
        scopy[(n_chunks - 1) & 1].wait()

    return gather_kernel


def kernel(embedding, sequence_ids):
    Bt, S = sequence_ids.shape
    V, D = embedding.shape
    tbl = jnp.zeros((64, D), jnp.float32).at[:V].set(embedding)
    left = jnp.broadcast_to(tbl[:, None, :], (64, 64, D))
    right = jnp.broadcast_to(tbl[None, :, :], (64, 64, D))
    tbl_pairs = jnp.concatenate([left, right], axis=-1).reshape(64 * 64, 2 * D)
    ids = sequence_ids.reshape(-1).astype(jnp.int32)
    info = plsc.get_sparse_core_info()
    out = _make_gather(ids.shape[0], info.num_cores, info.num_subcores)(
        tbl_pairs, ids
    )
    return out.reshape(Bt, S, D)


# speedup vs baseline: 6.6166x; 1.3252x over previous
import functools

import jax
import jax.numpy as jnp
from jax import lax
from jax.experimental import pallas as pl
from jax.experimental.pallas import tpu as pltpu
from jax.experimental.pallas import tpu_sc as plsc

_L = 16
_D = 64


def _make_tgather(S, Btot, NC, NS):
    NW = NC * NS
    BC = 512
    nbc = Btot // BC
    units = S * nbc
    per_w = units // NW
    last_u = units - 1

    mesh = plsc.VectorSubcoreMesh(core_axis_name="c", subcore_axis_name="s")

    @functools.partial(
        pl.kernel,
        mesh=mesh,
        out_type=jax.ShapeDtypeStruct((S * _D, Btot), jnp.float32),
        scratch_types=[
            pltpu.VMEM((_D * 64,), jnp.float32),
            pltpu.VMEM((BC,), jnp.int32),
            pltpu.VMEM((BC,), jnp.int32),
            pltpu.VMEM((_D, BC), jnp.float32),
            pltpu.VMEM((_D, BC), jnp.float32),
            pltpu.SemaphoreType.DMA,
            pltpu.SemaphoreType.DMA,
            pltpu.SemaphoreType.DMA,
            pltpu.SemaphoreType.DMA,
        ],
        compiler_params=pltpu.CompilerParams(needs_layout_passes=False),
    )
    def tgather_kernel(
        tbl_hbm, ids_hbm, out_hbm, tbl_v, idb0, idb1, blk0, blk1,
        si0, si1, so0, so1
    ):
        idb = [idb0, idb1]
        blk = [blk0, blk1]
        si = [si0, si1]
        so = [so0, so1]
        wid = lax.axis_index("s") * NC + lax.axis_index("c")
        u0 = wid * per_w
        pltpu.sync_copy(tbl_hbm, tbl_v)

        def ids_copy(u, b):
            return pltpu.make_async_copy(
                ids_hbm.at[pl.ds(u * BC, BC)], idb[b], si[b]
            )

        def out_copy(u, b):
            s = u // nbc
            bc = u - s * nbc
            return pltpu.make_async_copy(
                blk[b],
                out_hbm.at[pl.ds(s * _D, _D), pl.ds(bc * BC, BC)],
                so[b],
            )

        def compute(b):
            def grp(g, c):
                ids16 = idb[b][pl.ds(g * _L, _L)] & 63
                for d in range(_D):
                    blk[b][d, pl.ds(g * _L, _L)] = plsc.load_gather(
                        tbl_v, [ids16 + d * 64]
                    )
                return c

            lax.fori_loop(0, BC // _L, grp, 0)

        ids_copy(u0, 0).start()
        ids_copy(u0 + 1, 1).start()
        for b in range(2):
            u = u0 + b
            ids_copy(u, b).wait()
            compute(b)
            ids_copy(u + 2, b).start()
            out_copy(u, b).start()

        def pair(j, c):
            u2 = u0 + 2 * j
            for b in range(2):
                u = u2 + b
                ids_copy(u, b).wait()
                compute(b)
                ids_copy(jnp.minimum(u + 2, last_u), b).start()
                out_copy(u, b).wait()
                out_copy(u, b).start()
            return c

        lax.fori_loop(1, per_w // 2, pair, 0)

        for b in range(2):
            out_copy(u0 + b, b).wait()
            ids_copy(u0 + b, b).wait()

    return tgather_kernel


def kernel(embedding, sequence_ids):
    Bt, S = sequence_ids.shape
    V, D = embedding.shape
    tbl_pad = jnp.zeros((64, D), jnp.float32).at[:V].set(embedding)
    tbl_t = tbl_pad.T.reshape(-1)
    ids_t = sequence_ids.T.reshape(-1).astype(jnp.int32)
    info = plsc.get_sparse_core_info()
    out_t = _make_tgather(S, Bt, info.num_cores, info.num_subcores)(
        tbl_t, ids_t
    )
    return out_t.reshape(S, D, Bt).transpose(2, 0, 1)


# speedup vs baseline: 7.5614x; 1.1428x over previous
import functools

import jax
import jax.numpy as jnp
from jax import lax
from jax.experimental import pallas as pl
from jax.experimental.pallas import tpu as pltpu
from jax.experimental.pallas import tpu_sc as plsc

_L = 16
_D = 64


def _make_tgather(S, Btot, NC, NS):
    NW = NC * NS
    BC = 256
    nbc = Btot // BC
    units = S * nbc
    per_w = units // NW
    last_u = units - 1

    mesh = plsc.VectorSubcoreMesh(core_axis_name="c", subcore_axis_name="s")

    @functools.partial(
        pl.kernel,
        mesh=mesh,
        out_type=jax.ShapeDtypeStruct((S * _D, Btot), jnp.float32),
        scratch_types=[
            pltpu.VMEM((_D * 64 * _L,), jnp.float32),
            pltpu.VMEM((BC,), jnp.int32),
            pltpu.VMEM((BC,), jnp.int32),
            pltpu.VMEM((_D, BC), jnp.float32),
            pltpu.VMEM((_D, BC), jnp.float32),
            pltpu.SemaphoreType.DMA,
            pltpu.SemaphoreType.DMA,
            pltpu.SemaphoreType.DMA,
            pltpu.SemaphoreType.DMA,
        ],
        compiler_params=pltpu.CompilerParams(needs_layout_passes=False),
    )
    def tgather_kernel(
        tbl_hbm, ids_hbm, out_hbm, tbl_v, idb0, idb1, blk0, blk1,
        si0, si1, so0, so1
    ):
        idb = [idb0, idb1]
        blk = [blk0, blk1]
        si = [si0, si1]
        so = [so0, so1]
        wid = lax.axis_index("s") * NC + lax.axis_index("c")
        u0 = wid * per_w
        pltpu.sync_copy(tbl_hbm, tbl_v)
        lanes = lax.iota(jnp.int32, _L)

        def ids_copy(u, b):
            return pltpu.make_async_copy(
                ids_hbm.at[pl.ds(u * BC, BC)], idb[b], si[b]
            )

        def out_copy(u, b):
            s = u // nbc
            bc = u - s * nbc
            return pltpu.make_async_copy(
                blk[b],
                out_hbm.at[pl.ds(s * _D, _D), pl.ds(bc * BC, BC)],
                so[b],
            )

        def compute(b):
            def grp(g, c):
                ids16 = (idb[b][pl.ds(g * _L, _L)] & 63) * _L + lanes
                for d in range(_D):
                    blk[b][d, pl.ds(g * _L, _L)] = plsc.load_gather(
                        tbl_v, [ids16 + d * (64 * _L)]
                    )
                return c

            lax.fori_loop(0, BC // _L, grp, 0)

        ids_copy(u0, 0).start()
        ids_copy(u0 + 1, 1).start()
        for b in range(2):
            u = u0 + b
            ids_copy(u, b).wait()
            compute(b)
            ids_copy(u + 2, b).start()
            out_copy(u, b).start()

        def pair(j, c):
            u2 = u0 + 2 * j
            for b in range(2):
                u = u2 + b
                ids_copy(u, b).wait()
                compute(b)
                ids_copy(jnp.minimum(u + 2, last_u), b).start()
                out_copy(u, b).wait()
                out_copy(u, b).start()
            return c

        lax.fori_loop(1, per_w // 2, pair, 0)

        for b in range(2):
            out_copy(u0 + b, b).wait()
            ids_copy(u0 + b, b).wait()

    return tgather_kernel


def kernel(embedding, sequence_ids):
    Bt, S = sequence_ids.shape
    V, D = embedding.shape
    tbl_pad = jnp.zeros((64, D), jnp.float32).at[:V].set(embedding)
    tbl_t = jnp.broadcast_to(
        tbl_pad.T.reshape(-1)[:, None], (64 * D, _L)
    ).reshape(-1)
    ids_t = sequence_ids.T.reshape(-1).astype(jnp.int32)
    info = plsc.get_sparse_core_info()
    out_t = _make_tgather(S, Bt, info.num_cores, info.num_subcores)(
        tbl_t, ids_t
    )
    return out_t.reshape(S, D, Bt).transpose(2, 0, 1)


# speedup vs baseline: 14.8184x; 1.9597x over previous
import functools

import jax
import jax.numpy as jnp
from jax import lax
from jax.experimental import pallas as pl
from jax.experimental.pallas import tpu as pltpu
from jax.experimental.pallas import tpu_sc as plsc

_L = 16
_D = 64


def _make_tgather(S, Btot, NC, NS):
    NW = NC * NS
    BC = 256
    nbc = Btot // BC
    units = S * nbc
    per_w = units // NW
    last_u = units - 1

    mesh = plsc.VectorSubcoreMesh(core_axis_name="c", subcore_axis_name="s")

    @functools.partial(
        pl.kernel,
        mesh=mesh,
        out_type=jax.ShapeDtypeStruct((S * _D, Btot), jnp.float32),
        scratch_types=[
            pltpu.VMEM((_D * 64 * _L,), jnp.float32),
            pltpu.VMEM((BC,), jnp.int32),
            pltpu.VMEM((BC,), jnp.int32),
            pltpu.VMEM((_D, BC), jnp.float32),
            pltpu.VMEM((_D, BC), jnp.float32),
            pltpu.SemaphoreType.DMA,
            pltpu.SemaphoreType.DMA,
            pltpu.SemaphoreType.DMA,
            pltpu.SemaphoreType.DMA,
        ],
        compiler_params=pltpu.CompilerParams(needs_layout_passes=False),
    )
    def tgather_kernel(
        tbl_hbm, ids_hbm, out_hbm, tbl_v, idb0, idb1, blk0, blk1,
        si0, si1, so0, so1
    ):
        idb = [idb0, idb1]
        blk = [blk0, blk1]
        si = [si0, si1]
        so = [so0, so1]
        wid = lax.axis_index("s") * NC + lax.axis_index("c")
        u0 = wid * per_w
        pltpu.sync_copy(tbl_hbm, tbl_v)
        lanes = lax.iota(jnp.int32, _L)

        def ids_copy(u, b):
            return pltpu.make_async_copy(
                ids_hbm.at[pl.ds(u * BC, BC)], idb[b], si[b]
            )

        def out_copy(u, b):
            s = u // nbc
            bc = u - s * nbc
            return pltpu.make_async_copy(
                blk[b],
                out_hbm.at[pl.ds(s * _D, _D), pl.ds(bc * BC, BC)],
                so[b],
            )

        def compute(b):
            @plsc.parallel_loop(0, BC // _L, unroll=2)
            def grp(g):
                ids16 = (idb[b][pl.ds(g * _L, _L)] & 63) * _L + lanes
                for d in range(_D):
                    blk[b][d, pl.ds(g * _L, _L)] = plsc.load_gather(
                        tbl_v, [ids16 + d * (64 * _L)]
                    )

        ids_copy(u0, 0).start()
        ids_copy(u0 + 1, 1).start()
        for b in range(2):
            u = u0 + b
            ids_copy(u, b).wait()
            compute(b)
            ids_copy(u + 2, b).start()
            out_copy(u, b).start()

        def pair(j, c):
            u2 = u0 + 2 * j
            for b in range(2):
                u = u2 + b
                ids_copy(u, b).wait()
                compute(b)
                ids_copy(jnp.minimum(u + 2, last_u), b).start()
                out_copy(u, b).wait()
                out_copy(u, b).start()
            return c

        lax.fori_loop(1, per_w // 2, pair, 0)

        for b in range(2):
            out_copy(u0 + b, b).wait()
            ids_copy(u0 + b, b).wait()

    return tgather_kernel


def kernel(embedding, sequence_ids):
    Bt, S = sequence_ids.shape
    V, D = embedding.shape
    tbl_pad = jnp.zeros((64, D), jnp.float32).at[:V].set(embedding)
    tbl_t = jnp.broadcast_to(
        tbl_pad.T.reshape(-1)[:, None], (64 * D, _L)
    ).reshape(-1)
    ids_t = sequence_ids.T.reshape(-1).astype(jnp.int32)
    info = plsc.get_sparse_core_info()
    out_t = _make_tgather(S, Bt, info.num_cores, info.num_subcores)(
        tbl_t, ids_t
    )
    return out_t.reshape(S, D, Bt).transpose(2, 0, 1)


# speedup vs baseline: 20.4196x; 1.3780x over previous
import functools

import jax
import jax.numpy as jnp
from jax import lax
from jax.experimental import pallas as pl
from jax.experimental.pallas import tpu as pltpu
from jax.experimental.pallas import tpu_sc as plsc

_L = 16
_D = 64


def _make_tgather(S, Btot, NC, NS):
    NW = NC * NS
    BC = 256
    nbc = Btot // BC
    units = S * nbc
    per_w = units // NW
    last_u = units - 1

    mesh = plsc.VectorSubcoreMesh(core_axis_name="c", subcore_axis_name="s")

    @functools.partial(
        pl.kernel,
        mesh=mesh,
        out_type=jax.ShapeDtypeStruct((S * _D, Btot), jnp.float32),
        scratch_types=[
            pltpu.VMEM((_D * 64 * _L,), jnp.float32),
            pltpu.VMEM((BC,), jnp.int32),
            pltpu.VMEM((BC,), jnp.int32),
            pltpu.VMEM((_D, BC), jnp.float32),
            pltpu.VMEM((_D, BC), jnp.float32),
            pltpu.SemaphoreType.DMA,
            pltpu.SemaphoreType.DMA,
            pltpu.SemaphoreType.DMA,
            pltpu.SemaphoreType.DMA,
        ],
        compiler_params=pltpu.CompilerParams(needs_layout_passes=False),
    )
    def tgather_kernel(
        tbl_hbm, ids_hbm, out_hbm, tbl_v, idb0, idb1, blk0, blk1,
        si0, si1, so0, so1
    ):
        idb = [idb0, idb1]
        blk = [blk0, blk1]
        si = [si0, si1]
        so = [so0, so1]
        wid = lax.axis_index("s") * NC + lax.axis_index("c")
        u0 = wid * per_w
        pltpu.sync_copy(tbl_hbm, tbl_v)
        lanes = lax.iota(jnp.int32, _L)

        def ids_copy(u, b):
            return pltpu.make_async_copy(
                ids_hbm.at[pl.ds(u * BC, BC)], idb[b], si[b]
            )

        def out_copy(u, b):
            s = u // nbc
            bc = u - s * nbc
            return pltpu.make_async_copy(
                blk[b],
                out_hbm.at[pl.ds(s * _D, _D), pl.ds(bc * BC, BC)],
                so[b],
            )

        def compute(b):
            @plsc.parallel_loop(0, BC // _L, unroll=4)
            def grp(g):
                ids16 = (idb[b][pl.ds(g * _L, _L)] & 63) * _L + lanes
                for d in range(_D):
                    blk[b][d, pl.ds(g * _L, _L)] = plsc.load_gather(
                        tbl_v, [ids16 + d * (64 * _L)]
                    )

        ids_copy(u0, 0).start()
        ids_copy(u0 + 1, 1).start()
        for b in range(2):
            u = u0 + b
            ids_copy(u, b).wait()
            compute(b)
            ids_copy(u + 2, b).start()
            out_copy(u, b).start()

        def pair(j, c):
            u2 = u0 + 2 * j
            for b in range(2):
                u = u2 + b
                ids_copy(u, b).wait()
                compute(b)
                ids_copy(jnp.minimum(u + 2, last_u), b).start()
                out_copy(u, b).wait()
                out_copy(u, b).start()
            return c

        lax.fori_loop(1, per_w // 2, pair, 0)

        for b in range(2):
            out_copy(u0 + b, b).wait()
            ids_copy(u0 + b, b).wait()

    return tgather_kernel


def kernel(embedding, sequence_ids):
    Bt, S = sequence_ids.shape
    V, D = embedding.shape
    tbl_pad = jnp.zeros((64, D), jnp.float32).at[:V].set(embedding)
    tbl_t = jnp.broadcast_to(
        tbl_pad.T.reshape(-1)[:, None], (64 * D, _L)
    ).reshape(-1)
    ids_t = sequence_ids.T.reshape(-1).astype(jnp.int32)
    info = plsc.get_sparse_core_info()
    out_t = _make_tgather(S, Bt, info.num_cores, info.num_subcores)(
        tbl_t, ids_t
    )
    return out_t.reshape(S, D, Bt).transpose(2, 0, 1)
